# baseline (speedup 1.0000x reference)
import functools

import jax
import jax.numpy as jnp
from jax import lax
from jax.experimental import pallas as pl
from jax.experimental.pallas import tpu as pltpu
from jax.experimental.pallas import tpu_sc as plsc

_GRP = 8


def _make_sc_gather(Dp, B):
  info = plsc.get_sparse_core_info()
  NC, NS = info.num_cores, info.num_subcores
  NW = NC * NS
  assert B % NW == 0
  b_per_w = B // NW
  G = 32
  n_chunks = b_per_w // G
  mesh = plsc.VectorSubcoreMesh(core_axis_name="c", subcore_axis_name="s")

  @functools.partial(
      pl.kernel, mesh=mesh,
      out_type=jax.ShapeDtypeStruct((B, _GRP, Dp), jnp.float32),
      scratch_types=[
          pltpu.VMEM((b_per_w,), jnp.int32),
          pltpu.VMEM((G, _GRP, Dp), jnp.float32),
          pltpu.VMEM((G, _GRP, Dp), jnp.float32),
          pltpu.SemaphoreType.DMA,
          pltpu.SemaphoreType.DMA,
          pltpu.SemaphoreType.DMA,
      ],
  )
  def gather_k(table_hbm, idx_hbm, out_hbm, idx_v, g0, g1, sa, sb, wsem):
    wid = lax.axis_index("s") * NC + lax.axis_index("c")
    base = wid * b_per_w
    pltpu.sync_copy(idx_hbm.at[pl.ds(base, b_per_w)], idx_v)
    bufs = ((g0, sa), (g1, sb))

    def fire(c, buf, sem):
      for k in range(G // 16):
        v = idx_v[pl.ds(c * G + k * 16, 16)]
        for j in range(16):
          pltpu.async_copy(table_hbm.at[v[j]], buf.at[k * 16 + j], sem)

    fire(0, g0, sa)
    for c in range(n_chunks):
      buf, sem = bufs[c % 2]
      if c + 1 < n_chunks:
        nbuf, nsem = bufs[(c + 1) % 2]
        if c >= 1:
          pltpu.make_async_copy(table_hbm.at[pl.ds(0, G)], nbuf, wsem).wait()
        fire(c + 1, nbuf, nsem)
      pltpu.make_async_copy(table_hbm.at[pl.ds(0, G)], buf, sem).wait()
      pltpu.async_copy(buf, out_hbm.at[pl.ds(base + c * G, G)], wsem)
    pltpu.make_async_copy(table_hbm.at[pl.ds(0, G)], g0, wsem).wait()
    pltpu.make_async_copy(table_hbm.at[pl.ds(0, G)], g1, wsem).wait()

  return gather_k


def _tc_gather_body(n_rows, idx_ref, table_ref, out_ref, s0, s1, s2, s3):
  sems = (s0, s1, s2, s3)

  def body(c, _):
    for j in range(4):
      i = c * 4 + j
      r = idx_ref[i]
      pltpu.make_async_copy(table_ref.at[r], out_ref.at[i], sems[j]).start()
    return 0

  lax.fori_loop(0, n_rows // 4, body, 0, unroll=8)
  for s in sems:
    pltpu.make_async_copy(table_ref.at[pl.ds(0, n_rows // 4)],
                          out_ref.at[pl.ds(0, n_rows // 4)], s).wait()


def _tc_gather(table, idx, n_rows, D):
  return pl.pallas_call(
      functools.partial(_tc_gather_body, n_rows),
      in_specs=[
          pl.BlockSpec(memory_space=pltpu.SMEM),
          pl.BlockSpec(memory_space=pltpu.HBM),
      ],
      out_specs=pl.BlockSpec(memory_space=pltpu.VMEM),
      out_shape=jax.ShapeDtypeStruct((n_rows, D), jnp.float32),
      scratch_shapes=[pltpu.SemaphoreType.DMA] * 4,
  )(idx, table)


def _mlp_tail(blk, a_pad, g_pad, item,
              age_ref, gen_ref, at_ref, gt_ref,
              w1a_ref, w1b_ref, w1c_ref, b1_ref, w2_ref, b2_ref,
              w3_ref, b3_ref, out_ref):
  age = age_ref[0, 0, :]
  gen = gen_ref[0, 0, :]
  oa = (age[:, None] == lax.broadcasted_iota(jnp.int32, (blk, a_pad), 1)
        ).astype(jnp.float32)
  og = (gen[:, None] == lax.broadcasted_iota(jnp.int32, (blk, g_pad), 1)
        ).astype(jnp.float32)
  age_emb = jnp.dot(oa, at_ref[...], preferred_element_type=jnp.float32)
  gen_emb = jnp.dot(og, gt_ref[...], preferred_element_type=jnp.float32)
  h = (jnp.dot(item, w1a_ref[...], preferred_element_type=jnp.float32)
       + jnp.dot(age_emb, w1b_ref[...], preferred_element_type=jnp.float32)
       + jnp.dot(gen_emb, w1c_ref[...], preferred_element_type=jnp.float32)
       + b1_ref[...])
  h = jnp.maximum(h, 0.0)
  h = jnp.maximum(jnp.dot(h, w2_ref[...], preferred_element_type=jnp.float32)
                  + b2_ref[...], 0.0)
  out_ref[...] = (jnp.dot(h, w3_ref[...], preferred_element_type=jnp.float32)
                  + b3_ref[...])


def _mlp_body(blk, a_pad, g_pad,
              age_ref, gen_ref, sub_ref, grp_ref, at_ref, gt_ref,
              w1a_ref, w1b_ref, w1c_ref, b1_ref, w2_ref, b2_ref,
              w3_ref, b3_ref, out_ref):
  grp = grp_ref[...]
  sub = sub_ref[0, 0, :]
  osel = (sub[:, None] == lax.broadcasted_iota(jnp.int32, (blk, _GRP), 1)
          ).astype(jnp.float32)
  item = jnp.sum(grp * osel[:, :, None], axis=1)
  _mlp_tail(blk, a_pad, g_pad, item, age_ref, gen_ref, at_ref, gt_ref,
            w1a_ref, w1b_ref, w1c_ref, b1_ref, w2_ref, b2_ref,
            w3_ref, b3_ref, out_ref)


def _mlp_body_plain(blk, a_pad, g_pad,
                    age_ref, gen_ref, item_ref, at_ref, gt_ref,
                    w1a_ref, w1b_ref, w1c_ref, b1_ref, w2_ref, b2_ref,
                    w3_ref, b3_ref, out_ref):
  _mlp_tail(blk, a_pad, g_pad, item_ref[...], age_ref, gen_ref, at_ref,
            gt_ref, w1a_ref, w1b_ref, w1c_ref, b1_ref, w2_ref, b2_ref,
            w3_ref, b3_ref, out_ref)


def kernel(candidate_itemid, candidate_item_age, candidate_item_gender,
           item_emb_table, age_emb_table, gender_emb_table,
           W1, b1, W2, b2, W3, b3):
  B = candidate_itemid.shape[0]
  V, D = item_emb_table.shape
  A_V, A_D = age_emb_table.shape
  G_V, G_D = gender_emb_table.shape
  H1 = W1.shape[1]

  itemid = candidate_itemid.astype(jnp.int32)
  age = candidate_item_age.astype(jnp.int32)
  gen = candidate_item_gender.astype(jnp.int32)

  B_SC = 13312
  B_TC = B - B_SC
  gid = lax.shift_right_logical(itemid[:B_SC], 3)
  sub = jnp.bitwise_and(itemid[:B_SC], _GRP - 1)

  table3 = item_emb_table.reshape(V // _GRP, _GRP, D)
  grp = _make_sc_gather(D, B_SC)(table3, gid)
  item_tc = _tc_gather(item_emb_table, itemid[B_SC:], B_TC, D)

  BLK = 1024
  A_PAD = 128
  G_PAD = 8
  at_pad = jnp.zeros((A_PAD, A_D), jnp.float32).at[:A_V].set(age_emb_table)
  gt_pad = jnp.zeros((G_PAD, G_D), jnp.float32).at[:G_V].set(gender_emb_table)
  w1a = W1[:D]
  w1b = W1[D:D + A_D]
  w1c = W1[D + A_D:]
  wargs = (at_pad, gt_pad, w1a, w1b, w1c, b1.reshape(1, -1), W2,
           b2.reshape(1, -1), W3, b3.reshape(1, -1))

  full = lambda shape: pl.BlockSpec(shape, lambda i: (0,) * len(shape))
  wspecs = [
      full((A_PAD, A_D)),
      full((G_PAD, G_D)),
      full((D, H1)),
      full((A_D, H1)),
      full((G_D, H1)),
      full((1, H1)),
      full(W2.shape),
      full((1, W2.shape[1])),
      full(W3.shape),
      full((1, W3.shape[1])),
  ]
  idx_spec = pl.BlockSpec((1, 1, BLK), lambda i: (i, 0, 0))
  DO = W3.shape[1]

  nb1 = B_SC // BLK
  out_sc = pl.pallas_call(
      functools.partial(_mlp_body, BLK, A_PAD, G_PAD),
      grid=(nb1,),
      in_specs=[idx_spec, idx_spec, idx_spec,
                pl.BlockSpec((BLK, _GRP, D), lambda i: (i, 0, 0))] + wspecs,
      out_specs=pl.BlockSpec((BLK, DO), lambda i: (i, 0)),
      out_shape=jax.ShapeDtypeStruct((B_SC, DO), jnp.float32),
  )(age[:B_SC].reshape(nb1, 1, BLK), gen[:B_SC].reshape(nb1, 1, BLK),
    sub.reshape(nb1, 1, BLK), grp, *wargs)

  nb2 = B_TC // BLK
  out_tc = pl.pallas_call(
      functools.partial(_mlp_body_plain, BLK, A_PAD, G_PAD),
      grid=(nb2,),
      in_specs=[idx_spec, idx_spec,
                pl.BlockSpec((BLK, D), lambda i: (i, 0))] + wspecs,
      out_specs=pl.BlockSpec((BLK, DO), lambda i: (i, 0)),
      out_shape=jax.ShapeDtypeStruct((B_TC, DO), jnp.float32),
  )(age[B_SC:].reshape(nb2, 1, BLK), gen[B_SC:].reshape(nb2, 1, BLK),
    item_tc, *wargs)

  return jnp.concatenate([out_sc, out_tc], axis=0)


# speedup vs baseline: 1.0017x; 1.0017x over previous
import functools

import jax
import jax.numpy as jnp
from jax import lax
from jax.experimental import pallas as pl
from jax.experimental.pallas import tpu as pltpu
from jax.experimental.pallas import tpu_sc as plsc

_GRP = 8


def _make_sc_gather(Dp, B):
  info = plsc.get_sparse_core_info()
  NC, NS = info.num_cores, info.num_subcores
  NW = NC * NS
  assert B % NW == 0
  b_per_w = B // NW
  G = 32
  n_chunks = b_per_w // G
  mesh = plsc.VectorSubcoreMesh(core_axis_name="c", subcore_axis_name="s")

  @functools.partial(
      pl.kernel, mesh=mesh,
      out_type=jax.ShapeDtypeStruct((B, _GRP, Dp), jnp.float32),
      scratch_types=[
          pltpu.VMEM((b_per_w,), jnp.int32),
          pltpu.VMEM((G, _GRP, Dp), jnp.float32),
          pltpu.VMEM((G, _GRP, Dp), jnp.float32),
          pltpu.SemaphoreType.DMA,
          pltpu.SemaphoreType.DMA,
          pltpu.SemaphoreType.DMA,
      ],
  )
  def gather_k(table_hbm, idx_hbm, out_hbm, idx_v, g0, g1, sa, sb, wsem):
    wid = lax.axis_index("s") * NC + lax.axis_index("c")
    base = wid * b_per_w
    pltpu.sync_copy(idx_hbm.at[pl.ds(base, b_per_w)], idx_v)
    bufs = ((g0, sa), (g1, sb))

    def fire(c, buf, sem):
      for k in range(G // 16):
        v = idx_v[pl.ds(c * G + k * 16, 16)]
        for j in range(16):
          pltpu.async_copy(table_hbm.at[v[j]], buf.at[k * 16 + j], sem)

    fire(0, g0, sa)
    for c in range(n_chunks):
      buf, sem = bufs[c % 2]
      if c + 1 < n_chunks:
        nbuf, nsem = bufs[(c + 1) % 2]
        if c >= 1:
          pltpu.make_async_copy(table_hbm.at[pl.ds(0, G)], nbuf, wsem).wait()
        fire(c + 1, nbuf, nsem)
      pltpu.make_async_copy(table_hbm.at[pl.ds(0, G)], buf, sem).wait()
      pltpu.async_copy(buf, out_hbm.at[pl.ds(base + c * G, G)], wsem)
    pltpu.make_async_copy(table_hbm.at[pl.ds(0, G)], g0, wsem).wait()
    pltpu.make_async_copy(table_hbm.at[pl.ds(0, G)], g1, wsem).wait()

  return gather_k


def _tc_gather_body(n_rows, idx_ref, table_ref, out_ref, sem):
  def body(i, _):
    r = idx_ref[i]
    pltpu.make_async_copy(table_ref.at[r], out_ref.at[i], sem).start()
    return 0

  lax.fori_loop(0, n_rows, body, 0, unroll=16)
  pltpu.make_async_copy(table_ref.at[pl.ds(0, n_rows)], out_ref, sem).wait()


def _tc_gather(table, idx, n_rows, D):
  return pl.pallas_call(
      functools.partial(_tc_gather_body, n_rows),
      in_specs=[
          pl.BlockSpec(memory_space=pltpu.SMEM),
          pl.BlockSpec(memory_space=pltpu.HBM),
      ],
      out_specs=pl.BlockSpec(memory_space=pltpu.VMEM),
      out_shape=jax.ShapeDtypeStruct((n_rows, D), jnp.float32),
      scratch_shapes=[pltpu.SemaphoreType.DMA],
  )(idx, table)


def _mlp_tail(blk, a_pad, g_pad, item,
              age_ref, gen_ref, at_ref, gt_ref,
              w1a_ref, w1b_ref, w1c_ref, b1_ref, w2_ref, b2_ref,
              w3_ref, b3_ref, out_ref):
  age = age_ref[0, 0, :]
  gen = gen_ref[0, 0, :]
  oa = (age[:, None] == lax.broadcasted_iota(jnp.int32, (blk, a_pad), 1)
        ).astype(jnp.float32)
  og = (gen[:, None] == lax.broadcasted_iota(jnp.int32, (blk, g_pad), 1)
        ).astype(jnp.float32)
  age_emb = jnp.dot(oa, at_ref[...], preferred_element_type=jnp.float32)
  gen_emb = jnp.dot(og, gt_ref[...], preferred_element_type=jnp.float32)
  h = (jnp.dot(item, w1a_ref[...], preferred_element_type=jnp.float32)
       + jnp.dot(age_emb, w1b_ref[...], preferred_element_type=jnp.float32)
       + jnp.dot(gen_emb, w1c_ref[...], preferred_element_type=jnp.float32)
       + b1_ref[...])
  h = jnp.maximum(h, 0.0)
  h = jnp.maximum(jnp.dot(h, w2_ref[...], preferred_element_type=jnp.float32)
                  + b2_ref[...], 0.0)
  out_ref[...] = (jnp.dot(h, w3_ref[...], preferred_element_type=jnp.float32)
                  + b3_ref[...])


def _mlp_body(blk, a_pad, g_pad,
              age_ref, gen_ref, sub_ref, grp_ref, at_ref, gt_ref,
              w1a_ref, w1b_ref, w1c_ref, b1_ref, w2_ref, b2_ref,
              w3_ref, b3_ref, out_ref):
  grp = grp_ref[...]
  sub = sub_ref[0, 0, :]
  osel = (sub[:, None] == lax.broadcasted_iota(jnp.int32, (blk, _GRP), 1)
          ).astype(jnp.float32)
  item = jnp.sum(grp * osel[:, :, None], axis=1)
  _mlp_tail(blk, a_pad, g_pad, item, age_ref, gen_ref, at_ref, gt_ref,
            w1a_ref, w1b_ref, w1c_ref, b1_ref, w2_ref, b2_ref,
            w3_ref, b3_ref, out_ref)


def _mlp_body_plain(blk, a_pad, g_pad,
                    age_ref, gen_ref, item_ref, at_ref, gt_ref,
                    w1a_ref, w1b_ref, w1c_ref, b1_ref, w2_ref, b2_ref,
                    w3_ref, b3_ref, out_ref):
  _mlp_tail(blk, a_pad, g_pad, item_ref[...], age_ref, gen_ref, at_ref,
            gt_ref, w1a_ref, w1b_ref, w1c_ref, b1_ref, w2_ref, b2_ref,
            w3_ref, b3_ref, out_ref)


def kernel(candidate_itemid, candidate_item_age, candidate_item_gender,
           item_emb_table, age_emb_table, gender_emb_table,
           W1, b1, W2, b2, W3, b3):
  B = candidate_itemid.shape[0]
  V, D = item_emb_table.shape
  A_V, A_D = age_emb_table.shape
  G_V, G_D = gender_emb_table.shape
  H1 = W1.shape[1]

  itemid = candidate_itemid.astype(jnp.int32)
  age = candidate_item_age.astype(jnp.int32)
  gen = candidate_item_gender.astype(jnp.int32)

  B_SC = 13312
  B_TC = B - B_SC
  gid = lax.shift_right_logical(itemid[:B_SC], 3)
  sub = jnp.bitwise_and(itemid[:B_SC], _GRP - 1)

  table3 = item_emb_table.reshape(V // _GRP, _GRP, D)
  grp = _make_sc_gather(D, B_SC)(table3, gid)
  item_tc = _tc_gather(item_emb_table, itemid[B_SC:], B_TC, D)

  BLK = 1024
  A_PAD = 128
  G_PAD = 8
  at_pad = jnp.zeros((A_PAD, A_D), jnp.float32).at[:A_V].set(age_emb_table)
  gt_pad = jnp.zeros((G_PAD, G_D), jnp.float32).at[:G_V].set(gender_emb_table)
  w1a = W1[:D]
  w1b = W1[D:D + A_D]
  w1c = W1[D + A_D:]
  wargs = (at_pad, gt_pad, w1a, w1b, w1c, b1.reshape(1, -1), W2,
           b2.reshape(1, -1), W3, b3.reshape(1, -1))

  full = lambda shape: pl.BlockSpec(shape, lambda i: (0,) * len(shape))
  wspecs = [
      full((A_PAD, A_D)),
      full((G_PAD, G_D)),
      full((D, H1)),
      full((A_D, H1)),
      full((G_D, H1)),
      full((1, H1)),
      full(W2.shape),
      full((1, W2.shape[1])),
      full(W3.shape),
      full((1, W3.shape[1])),
  ]
  idx_spec = pl.BlockSpec((1, 1, BLK), lambda i: (i, 0, 0))
  DO = W3.shape[1]

  nb1 = B_SC // BLK
  out_sc = pl.pallas_call(
      functools.partial(_mlp_body, BLK, A_PAD, G_PAD),
      grid=(nb1,),
      in_specs=[idx_spec, idx_spec, idx_spec,
                pl.BlockSpec((BLK, _GRP, D), lambda i: (i, 0, 0))] + wspecs,
      out_specs=pl.BlockSpec((BLK, DO), lambda i: (i, 0)),
      out_shape=jax.ShapeDtypeStruct((B_SC, DO), jnp.float32),
  )(age[:B_SC].reshape(nb1, 1, BLK), gen[:B_SC].reshape(nb1, 1, BLK),
    sub.reshape(nb1, 1, BLK), grp, *wargs)

  nb2 = B_TC // BLK
  out_tc = pl.pallas_call(
      functools.partial(_mlp_body_plain, BLK, A_PAD, G_PAD),
      grid=(nb2,),
      in_specs=[idx_spec, idx_spec,
                pl.BlockSpec((BLK, D), lambda i: (i, 0))] + wspecs,
      out_specs=pl.BlockSpec((BLK, DO), lambda i: (i, 0)),
      out_shape=jax.ShapeDtypeStruct((B_TC, DO), jnp.float32),
  )(age[B_SC:].reshape(nb2, 1, BLK), gen[B_SC:].reshape(nb2, 1, BLK),
    item_tc, *wargs)

  return jnp.concatenate([out_sc, out_tc], axis=0)


# speedup vs baseline: 1.3482x; 1.3459x over previous
import functools

import jax
import jax.numpy as jnp
from jax import lax
from jax.experimental import pallas as pl
from jax.experimental.pallas import tpu as pltpu
from jax.experimental.pallas import tpu_sc as plsc

_GRP = 8


def _make_sc_gather(Dp, B):
  info = plsc.get_sparse_core_info()
  NC, NS = info.num_cores, info.num_subcores
  NW = NC * NS
  assert B % NW == 0
  b_per_w = B // NW
  G = 32
  n_chunks = b_per_w // G
  mesh = plsc.VectorSubcoreMesh(core_axis_name="c", subcore_axis_name="s")

  @functools.partial(
      pl.kernel, mesh=mesh,
      out_type=jax.ShapeDtypeStruct((B, _GRP, Dp), jnp.float32),
      scratch_types=[
          pltpu.VMEM((b_per_w,), jnp.int32),
          pltpu.VMEM((G, _GRP, Dp), jnp.float32),
          pltpu.VMEM((G, _GRP, Dp), jnp.float32),
          pltpu.SemaphoreType.DMA,
          pltpu.SemaphoreType.DMA,
          pltpu.SemaphoreType.DMA,
      ],
  )
  def gather_k(table_hbm, idx_hbm, out_hbm, idx_v, g0, g1, sa, sb, wsem):
    wid = lax.axis_index("s") * NC + lax.axis_index("c")
    base = wid * b_per_w
    pltpu.sync_copy(idx_hbm.at[pl.ds(base, b_per_w)], idx_v)
    bufs = ((g0, sa), (g1, sb))

    def fire(c, buf, sem):
      for k in range(G // 16):
        v = idx_v[pl.ds(c * G + k * 16, 16)]
        for j in range(16):
          pltpu.async_copy(table_hbm.at[v[j]], buf.at[k * 16 + j], sem)

    fire(0, g0, sa)
    for c in range(n_chunks):
      buf, sem = bufs[c % 2]
      if c + 1 < n_chunks:
        nbuf, nsem = bufs[(c + 1) % 2]
        if c >= 1:
          pltpu.make_async_copy(table_hbm.at[pl.ds(0, G)], nbuf, wsem).wait()
        fire(c + 1, nbuf, nsem)
      pltpu.make_async_copy(table_hbm.at[pl.ds(0, G)], buf, sem).wait()
      pltpu.async_copy(buf, out_hbm.at[pl.ds(base + c * G, G)], wsem)
    pltpu.make_async_copy(table_hbm.at[pl.ds(0, G)], g0, wsem).wait()
    pltpu.make_async_copy(table_hbm.at[pl.ds(0, G)], g1, wsem).wait()

  return gather_k


def _mlp_body(blk, a_pad, g_pad,
              age_ref, gen_ref, sub_ref, grp_ref, at_ref, gt_ref,
              w1a_ref, w1b_ref, w1c_ref, b1_ref, w2_ref, b2_ref,
              w3_ref, b3_ref, out_ref):
  grp = grp_ref[...]
  sub = sub_ref[0, 0, :]
  osel = (sub[:, None] == lax.broadcasted_iota(jnp.int32, (blk, _GRP), 1)
          ).astype(jnp.float32)
  item = jnp.sum(grp * osel[:, :, None], axis=1)
  age = age_ref[0, 0, :]
  gen = gen_ref[0, 0, :]
  oa = (age[:, None] == lax.broadcasted_iota(jnp.int32, (blk, a_pad), 1)
        ).astype(jnp.float32)
  og = (gen[:, None] == lax.broadcasted_iota(jnp.int32, (blk, g_pad), 1)
        ).astype(jnp.float32)
  age_emb = jnp.dot(oa, at_ref[...], preferred_element_type=jnp.float32)
  gen_emb = jnp.dot(og, gt_ref[...], preferred_element_type=jnp.float32)
  h = (jnp.dot(item, w1a_ref[...], preferred_element_type=jnp.float32)
       + jnp.dot(age_emb, w1b_ref[...], preferred_element_type=jnp.float32)
       + jnp.dot(gen_emb, w1c_ref[...], preferred_element_type=jnp.float32)
       + b1_ref[...])
  h = jnp.maximum(h, 0.0)
  h = jnp.maximum(jnp.dot(h, w2_ref[...], preferred_element_type=jnp.float32)
                  + b2_ref[...], 0.0)
  out_ref[...] = (jnp.dot(h, w3_ref[...], preferred_element_type=jnp.float32)
                  + b3_ref[...])


def kernel(candidate_itemid, candidate_item_age, candidate_item_gender,
           item_emb_table, age_emb_table, gender_emb_table,
           W1, b1, W2, b2, W3, b3):
  B = candidate_itemid.shape[0]
  V, D = item_emb_table.shape
  A_V, A_D = age_emb_table.shape
  G_V, G_D = gender_emb_table.shape
  H1 = W1.shape[1]

  itemid = candidate_itemid.astype(jnp.int32)
  age = candidate_item_age.astype(jnp.int32)
  gen = candidate_item_gender.astype(jnp.int32)
  gid = lax.shift_right_logical(itemid, 3)
  sub = jnp.bitwise_and(itemid, _GRP - 1)

  table3 = item_emb_table.reshape(V // _GRP, _GRP, D)
  grp = _make_sc_gather(D, B)(table3, gid)

  BLK = 1024
  NB = B // BLK
  A_PAD = 128
  G_PAD = 8
  at_pad = jnp.zeros((A_PAD, A_D), jnp.float32).at[:A_V].set(age_emb_table)
  gt_pad = jnp.zeros((G_PAD, G_D), jnp.float32).at[:G_V].set(gender_emb_table)
  w1a = W1[:D]
  w1b = W1[D:D + A_D]
  w1c = W1[D + A_D:]
  age3 = age.reshape(NB, 1, BLK)
  gen3 = gen.reshape(NB, 1, BLK)
  sub3 = sub.reshape(NB, 1, BLK)

  full = lambda shape: pl.BlockSpec(shape, lambda i: (0,) * len(shape))
  out = pl.pallas_call(
      functools.partial(_mlp_body, BLK, A_PAD, G_PAD),
      grid=(NB,),
      in_specs=[
          pl.BlockSpec((1, 1, BLK), lambda i: (i, 0, 0)),
          pl.BlockSpec((1, 1, BLK), lambda i: (i, 0, 0)),
          pl.BlockSpec((1, 1, BLK), lambda i: (i, 0, 0)),
          pl.BlockSpec((BLK, _GRP, D), lambda i: (i, 0, 0)),
          full((A_PAD, A_D)),
          full((G_PAD, G_D)),
          full((D, H1)),
          full((A_D, H1)),
          full((G_D, H1)),
          full((1, H1)),
          full(W2.shape),
          full((1, W2.shape[1])),
          full(W3.shape),
          full((1, W3.shape[1])),
      ],
      out_specs=pl.BlockSpec((BLK, W3.shape[1]), lambda i: (i, 0)),
      out_shape=jax.ShapeDtypeStruct((B, W3.shape[1]), jnp.float32),
  )(age3, gen3, sub3, grp, at_pad, gt_pad,
    w1a, w1b, w1c, b1.reshape(1, -1), W2, b2.reshape(1, -1),
    W3, b3.reshape(1, -1))
  return out
